# Initial kernel scaffold; baseline (speedup 1.0000x reference)
import jax
import jax.numpy as jnp
from jax.experimental import pallas as pl


def kernel(x, edge_index, Wz, bz, Lz, lbz, Wr, br, Lr, lbr, Wh, bh, Lh, lbh, l1w, l1b, bng, bnb, l2w, l2b):
    raise NotImplementedError("write your pallas kernel here")


# speedup vs baseline: 260.7597x; 260.7597x over previous
import functools

import jax
import jax.numpy as jnp
from jax import lax
from jax.experimental import pallas as pl
from jax.experimental.pallas import tpu as pltpu
from jax.experimental.pallas import tpu_sc as plsc

_N = 10000
_E = 320000
_NW = 32
_CHUNK = _E // _NW
_LANES = 16


def _sc_hist_body(src_hbm, dst_hbm, deg_out, match_out, src_v, dst_v, dh, mh):
    c = lax.axis_index("c")
    s = lax.axis_index("s")
    wid = s * 2 + c
    base = wid * _CHUNK
    pltpu.sync_copy(src_hbm.at[pl.ds(base, _CHUNK)], src_v)
    pltpu.sync_copy(dst_hbm.at[pl.ds(base, _CHUNK)], dst_v)

    zeros = jnp.zeros((_LANES,), jnp.int32)
    ones = jnp.ones((_LANES,), jnp.int32)
    two = jnp.full((_LANES,), 2, jnp.int32)

    def zero_body(i, _):
        dh[pl.ds(i * _LANES, _LANES)] = zeros
        mh[pl.ds(i * _LANES, _LANES)] = zeros
        return 0

    lax.fori_loop(0, _N // _LANES, zero_body, 0)

    def hist_body(i, _):
        d = dst_v[pl.ds(i * _LANES, _LANES)]
        sv = src_v[pl.ds(i * _LANES, _LANES)]
        plsc.addupdate_scatter(dh, [d], ones)
        plsc.addupdate_scatter(mh, [sv], ones, mask=d == two)
        return 0

    lax.fori_loop(0, _CHUNK // _LANES, hist_body, 0)

    pltpu.sync_copy(dh, deg_out.at[wid])
    pltpu.sync_copy(mh, match_out.at[wid])


_sc_hist = functools.partial(
    pl.kernel,
    out_type=(
        jax.ShapeDtypeStruct((_NW, _N), jnp.int32),
        jax.ShapeDtypeStruct((_NW, _N), jnp.int32),
    ),
    mesh=plsc.VectorSubcoreMesh(core_axis_name="c", subcore_axis_name="s"),
    compiler_params=pltpu.CompilerParams(needs_layout_passes=False),
    scratch_types=[
        pltpu.VMEM((_CHUNK,), jnp.int32),
        pltpu.VMEM((_CHUNK,), jnp.int32),
        pltpu.VMEM((_N,), jnp.int32),
        pltpu.VMEM((_N,), jnp.int32),
    ],
)(_sc_hist_body)


def _tc_head_body(deg_ref, match_ref, x_ref, wz_ref, bz_ref, lz_ref, lbz_ref,
                  wh_ref, bh_ref, lh_ref, lbh_ref, l1w_ref, l1b_ref, bng_ref,
                  bnb_ref, l2w_ref, l2b_ref, out_ref):
    deg = jnp.sum(deg_ref[...].astype(jnp.float32), axis=0, keepdims=True) + 1.0
    dinv = lax.rsqrt(deg)
    mc = jnp.sum(match_ref[...].astype(jnp.float32), axis=0, keepdims=True)
    lane = lax.broadcasted_iota(jnp.int32, (1, _N), 1)
    mc = mc + (lane == 2).astype(jnp.float32)
    dinv2 = lax.slice(dinv, (0, 2), (1, 3))
    w = mc * dinv * dinv2
    v = jnp.dot(w, x_ref[...], preferred_element_type=jnp.float32)
    cz = jnp.dot(v, wz_ref[...], preferred_element_type=jnp.float32) + bz_ref[...]
    ch = jnp.dot(v, wh_ref[...], preferred_element_type=jnp.float32) + bh_ref[...]
    z = jax.nn.sigmoid(
        jnp.dot(cz, lz_ref[...], preferred_element_type=jnp.float32) + lbz_ref[...])
    ht = jnp.tanh(
        jnp.dot(ch, lh_ref[...], preferred_element_type=jnp.float32) + lbh_ref[...])
    y = jax.nn.relu((1.0 - z) * ht)
    y = jnp.dot(y, l1w_ref[...], preferred_element_type=jnp.float32) + l1b_ref[...]
    y = y * (bng_ref[...] / jnp.sqrt(1.0 + 1e-5)) + bnb_ref[...]
    y = jax.nn.relu(y)
    out_ref[...] = jnp.dot(y, l2w_ref[...], preferred_element_type=jnp.float32) + l2b_ref[...]


def kernel(x, edge_index, Wz, bz, Lz, lbz, Wr, br, Lr, lbr, Wh, bh, Lh, lbh,
           l1w, l1b, bng, bnb, l2w, l2b):
    src = edge_index[0]
    dst = edge_index[1]
    deg_parts, match_parts = _sc_hist(src, dst)
    out = pl.pallas_call(
        _tc_head_body,
        out_shape=jax.ShapeDtypeStruct((1, 1), jnp.float32),
    )(deg_parts, match_parts, x,
      Wz, bz.reshape(1, -1), Lz[:128], lbz.reshape(1, -1),
      Wh, bh.reshape(1, -1), Lh[:128], lbh.reshape(1, -1),
      l1w, l1b.reshape(1, -1), bng.reshape(1, -1), bnb.reshape(1, -1),
      l2w, l2b.reshape(1, -1))
    return out.reshape((1,))
